# baseline (speedup 1.0000x reference)
import functools

import jax
import jax.numpy as jnp
from jax import lax
from jax.experimental import pallas as pl
from jax.experimental.pallas import tpu as pltpu
from jax.experimental.pallas import tpu_sc as plsc

L = 16
UNROLL = 8
ROWS = 2
NBUF = 3


def _sc_pe_add(B, S_sc, D, s_off):
    NC, NS = 2, 16
    NW = NC * NS
    sw = S_sc // NW
    n_chunks = sw // ROWS
    PEEL = 2
    K = (n_chunks - PEEL) // NBUF
    assert PEEL + K * NBUF == n_chunks

    mesh = plsc.VectorSubcoreMesh(core_axis_name="c", subcore_axis_name="s")

    @functools.partial(
        pl.kernel,
        out_type=jax.ShapeDtypeStruct((B, S_sc, D), jnp.float32),
        mesh=mesh,
        scratch_types=(
            [pltpu.VMEM((B, ROWS, D), jnp.float32) for _ in range(NBUF)]
            + [pltpu.VMEM((ROWS, D), jnp.float32) for _ in range(NBUF)]
            + [pltpu.SemaphoreType.DMA for _ in range(3 * NBUF)]
        ),
    )
    def body(x_hbm, pe_hbm, out_hbm, *scratch):
        xa = scratch[0:NBUF]
        pe_v = scratch[NBUF:2 * NBUF]
        sem_ld = scratch[2 * NBUF:3 * NBUF]
        sem_pe = scratch[3 * NBUF:4 * NBUF]
        sem_st = scratch[4 * NBUF:5 * NBUF]

        wid = lax.axis_index("s") * NC + lax.axis_index("c")
        base = wid * sw

        def issue_loads(j, sl):
            sg = s_off + sl
            pltpu.async_copy(pe_hbm.at[pl.ds(sg, ROWS)], pe_v[j], sem_pe[j])
            pltpu.async_copy(x_hbm.at[:, pl.ds(sg, ROWS)], xa[j], sem_ld[j])

        def run_chunk(j, jprev, sl, drain_prev, prefetch, k=None):
            sg = s_off + sl
            pltpu.make_async_copy(
                pe_hbm.at[pl.ds(sg, ROWS)], pe_v[j], sem_pe[j]
            ).wait()
            pltpu.make_async_copy(
                x_hbm.at[:, pl.ds(sg, ROWS)], xa[j], sem_ld[j]
            ).wait()

            @plsc.parallel_loop(0, D // L, unroll=UNROLL)
            def cbody(i):
                off = i * L
                for r in range(ROWS):
                    p = pe_v[j][r, pl.ds(off, L)]
                    for b in range(B):
                        xa[j][b, r, pl.ds(off, L)] = (
                            xa[j][b, r, pl.ds(off, L)] + p
                        )

            pltpu.async_copy(xa[j], out_hbm.at[:, pl.ds(sl, ROWS)], sem_st[j])

            if drain_prev:
                pltpu.make_async_copy(
                    xa[jprev], out_hbm.at[:, pl.ds(sl, ROWS)], sem_st[jprev]
                ).wait()

            if prefetch == "always":
                issue_loads(jprev, sl + 2 * ROWS)
            elif prefetch == "guarded":
                @pl.when(k < K - 1)
                def _():
                    issue_loads(jprev, sl + 2 * ROWS)

        issue_loads(0, base)
        issue_loads(1, base + ROWS)
        run_chunk(0, NBUF - 1, base, drain_prev=False, prefetch="always")
        run_chunk(1, 0, base + ROWS, drain_prev=True, prefetch="always")

        def step(k, carry):
            for m in range(NBUF):
                c = PEEL + NBUF * k + m
                j = (PEEL + m) % NBUF
                jprev = (j - 1) % NBUF
                sl = base + c * ROWS
                guard = "always" if m == 0 else "guarded"
                run_chunk(j, jprev, sl, drain_prev=True, prefetch=guard, k=k)
            return carry

        lax.fori_loop(0, K, step, 0)

        jlast = (n_chunks - 1) % NBUF
        pltpu.make_async_copy(
            xa[jlast], out_hbm.at[:, pl.ds(base, ROWS)], sem_st[jlast]
        ).wait()

    return body


def kernel(x, pe_table):
    B, S, D = x.shape
    return _sc_pe_add(B, S, D, 0)(x, pe_table)


# speedup vs baseline: 1.3079x; 1.3079x over previous
import jax
import jax.numpy as jnp
from jax.experimental import pallas as pl

TC_BS = 256


def _tc_add_body(x_ref, pe_ref, o_ref):
    o_ref[...] = x_ref[...] + pe_ref[...]


def kernel(x, pe_table):
    B, S, D = x.shape
    return pl.pallas_call(
        _tc_add_body,
        grid=(S // TC_BS, B),
        in_specs=[
            pl.BlockSpec((1, TC_BS, D), lambda i, b: (b, i, 0)),
            pl.BlockSpec((TC_BS, D), lambda i, b: (i, 0)),
        ],
        out_specs=pl.BlockSpec((1, TC_BS, D), lambda i, b: (b, i, 0)),
        out_shape=jax.ShapeDtypeStruct((B, S, D), x.dtype),
    )(x, pe_table)


# speedup vs baseline: 1.3100x; 1.0016x over previous
import jax
import jax.numpy as jnp
from jax.experimental import pallas as pl
from jax.experimental.pallas import tpu as pltpu

TC_BS = 256


def _tc_add_body(x_ref, pe_ref, o_ref):
    o_ref[...] = x_ref[...] + pe_ref[...]


def kernel(x, pe_table):
    B, S, D = x.shape
    return pl.pallas_call(
        _tc_add_body,
        grid=(S // TC_BS, B),
        in_specs=[
            pl.BlockSpec((1, TC_BS, D), lambda i, b: (b, i, 0)),
            pl.BlockSpec((TC_BS, D), lambda i, b: (i, 0)),
        ],
        out_specs=pl.BlockSpec((1, TC_BS, D), lambda i, b: (b, i, 0)),
        out_shape=jax.ShapeDtypeStruct((B, S, D), x.dtype),
        compiler_params=pltpu.CompilerParams(
            dimension_semantics=("parallel", "parallel"),
        ),
    )(x, pe_table)


# speedup vs baseline: 1.3650x; 1.0420x over previous
import jax
import jax.numpy as jnp
from jax.experimental import pallas as pl
from jax.experimental.pallas import tpu as pltpu

TC_BS = 512


def _tc_add_body(x_ref, pe_ref, o_ref):
    o_ref[...] = x_ref[...] + pe_ref[...]


def kernel(x, pe_table):
    B, S, D = x.shape
    return pl.pallas_call(
        _tc_add_body,
        grid=(S // TC_BS, B),
        in_specs=[
            pl.BlockSpec((1, TC_BS, D), lambda i, b: (b, i, 0)),
            pl.BlockSpec((TC_BS, D), lambda i, b: (i, 0)),
        ],
        out_specs=pl.BlockSpec((1, TC_BS, D), lambda i, b: (b, i, 0)),
        out_shape=jax.ShapeDtypeStruct((B, S, D), x.dtype),
        compiler_params=pltpu.CompilerParams(
            dimension_semantics=("parallel", "parallel"),
        ),
    )(x, pe_table)
